# baseline (speedup 1.0000x reference)
import jax
import jax.numpy as jnp
import numpy as np
from jax import lax
from jax.experimental import pallas as pl
from jax.experimental.pallas import tpu as pltpu
from jax.experimental.pallas import tpu_sc as plsc

NC = 2
NS = 16
CK = 128
NBUF = 2

_HALF_PERM = np.concatenate(
    [np.arange(0, 32, 2), np.arange(1, 32, 2),
     32 + np.arange(0, 32, 2), 32 + np.arange(1, 32, 2)])
_FULL_PERM = np.concatenate([_HALF_PERM, 64 + _HALF_PERM])


def _sc_segment_sum(n_pad, rows_per_sub, ch):
  mesh = plsc.VectorSubcoreMesh(core_axis_name="c", subcore_axis_name="s")

  def body(xlo_hbm, xhi_hbm, src_hbm, dst_hbm, outlo_hbm, outhi_hbm,
           zbuf, srcv, dstv, rb0, rb1, fr0, fr1, acc, sg0, sg1, ss0, ss1):
    rb = (rb0, rb1)
    fr = (fr0, fr1)
    sg = (sg0, sg1)
    ss = (ss0, ss1)
    cid = lax.axis_index("c")
    sid = lax.axis_index("s")

    zero16 = jnp.zeros((16,), jnp.float32)
    zrows = rows_per_sub // 8

    def zbody(i, _):
      for j in range(4):
        zbuf[i, pl.ds(j * 16, 16)] = zero16
      return 0

    lax.fori_loop(0, zrows, zbody, 0)
    for j in range(8):
      pltpu.sync_copy(zbuf, acc.at[pl.ds(sid * rows_per_sub + j * zrows, zrows)])
    plsc.subcore_barrier()

    pltpu.sync_copy(src_hbm.at[sid], srcv)
    pltpu.sync_copy(dst_hbm.at[sid], dstv)

    mask_hi = jnp.full((16,), -65536, jnp.int32)

    def expand(b):
      rbb, frb = rb[b], fr[b]

      def conv(q, _):
        r = q * 4
        for k in range(4):
          rk = r + k
          for j in range(2):
            w = rbb[rk, pl.ds(j * 16, 16)]
            frb[rk, pl.ds(j * 32, 16)] = lax.bitcast_convert_type(
                w << 16, jnp.float32)
            frb[rk, pl.ds(j * 32 + 16, 16)] = lax.bitcast_convert_type(
                w & mask_hi, jnp.float32)
        return 0

      lax.fori_loop(0, CK // 4, conv, 0)

    def run(x_ref):
      for b in range(NBUF):
        pltpu.async_copy(x_ref.at[srcv.at[b]], rb[b], sg[b])

      for b in range(NBUF):
        pltpu.make_async_copy(x_ref.at[srcv.at[b]], rb[b], sg[b]).wait()
        expand(b)
        pltpu.async_copy(fr[b], acc.at[dstv.at[b]], ss[b], add=True)
        pltpu.async_copy(x_ref.at[srcv.at[b + NBUF]], rb[b], sg[b])

      def gbody(g, _):
        c0 = g * NBUF
        for b in range(NBUF):
          c = c0 + b
          pltpu.make_async_copy(x_ref.at[srcv.at[c]], rb[b], sg[b]).wait()
          pltpu.make_async_copy(fr[b], acc.at[dstv.at[c - NBUF]], ss[b]).wait()
          expand(b)
          pltpu.async_copy(fr[b], acc.at[dstv.at[c]], ss[b], add=True)
          pltpu.async_copy(x_ref.at[srcv.at[c + NBUF]], rb[b], sg[b])
        return 0

      lax.fori_loop(1, ch // NBUF - 1, gbody, 0)

      c0 = ch - NBUF
      for b in range(NBUF):
        c = c0 + b
        pltpu.make_async_copy(x_ref.at[srcv.at[c]], rb[b], sg[b]).wait()
        pltpu.make_async_copy(fr[b], acc.at[dstv.at[c - NBUF]], ss[b]).wait()
        expand(b)
        pltpu.async_copy(fr[b], acc.at[dstv.at[c]], ss[b], add=True)
      for b in range(NBUF):
        pltpu.make_async_copy(fr[b], acc.at[dstv.at[c0 + b]], ss[b]).wait()

    pl.when(cid == 0)(lambda: run(xlo_hbm))
    pl.when(cid == 1)(lambda: run(xhi_hbm))
    plsc.subcore_barrier()

    sl = pl.ds(sid * rows_per_sub, rows_per_sub)
    pl.when(cid == 0)(lambda: pltpu.sync_copy(acc.at[sl], outlo_hbm.at[sl]))
    pl.when(cid == 1)(lambda: pltpu.sync_copy(acc.at[sl], outhi_hbm.at[sl]))

  return pl.kernel(
      body,
      out_type=(
          jax.ShapeDtypeStruct((n_pad, 64), jnp.float32),
          jax.ShapeDtypeStruct((n_pad, 64), jnp.float32),
      ),
      mesh=mesh,
      compiler_params=pltpu.CompilerParams(use_tc_tiling_on_sc=False),
      scratch_types=[
          pltpu.VMEM((rows_per_sub // 8, 64), jnp.float32),
          pltpu.VMEM((ch, CK), jnp.int32),
          pltpu.VMEM((ch, CK), jnp.int32),
          pltpu.VMEM((CK, 32), jnp.int32),
          pltpu.VMEM((CK, 32), jnp.int32),
          pltpu.VMEM((CK, 64), jnp.float32),
          pltpu.VMEM((CK, 64), jnp.float32),
          pltpu.VMEM_SHARED((n_pad, 64), jnp.float32),
          pltpu.SemaphoreType.DMA,
          pltpu.SemaphoreType.DMA,
          pltpu.SemaphoreType.DMA,
          pltpu.SemaphoreType.DMA,
      ],
  )


def _tc_body(plo_ref, phi_ref, w_ref, b_ref, o_ref):
  o_ref[...] = (
      jnp.dot(plo_ref[...], w_ref[0:64, :], preferred_element_type=jnp.float32)
      + jnp.dot(phi_ref[...], w_ref[64:128, :], preferred_element_type=jnp.float32)
      + b_ref[...]
  )


def _tc_combine_matmul(plo, phi, weight_perm, bias, n):
  br = 1000
  return pl.pallas_call(
      _tc_body,
      grid=(n // br,),
      in_specs=[
          pl.BlockSpec((br, 64), lambda i: (i, 0)),
          pl.BlockSpec((br, 64), lambda i: (i, 0)),
          pl.BlockSpec((128, 128), lambda i: (0, 0)),
          pl.BlockSpec((1, 128), lambda i: (0, 0)),
      ],
      out_specs=pl.BlockSpec((br, 128), lambda i: (i, 0)),
      out_shape=jax.ShapeDtypeStruct((n, 128), jnp.float32),
  )(plo, phi, weight_perm, bias.reshape(1, 128))


@jax.jit
def kernel(x, edge_index, weight, bias):
  n, d = x.shape
  e = edge_index.shape[1]
  assert d == 128 and weight.shape == (128, 128)

  ch = NBUF * (-(-e // (NS * CK * NBUF)))
  e_pad = NS * ch * CK
  rows_per_sub = 8 * (-(-(n + 1) // (NS * 8)))
  n_pad = rows_per_sub * NS

  src = edge_index[0]
  dst = edge_index[1]
  pad = e_pad - e
  src_p = jnp.concatenate([src, jnp.zeros((pad,), jnp.int32)]).reshape(NS, ch, CK)
  dst_p = jnp.concatenate([dst, jnp.full((pad,), n, jnp.int32)]).reshape(NS, ch, CK)

  xb = x.astype(jnp.bfloat16)
  x_lo = jax.lax.bitcast_convert_type(xb[:, :64].reshape(n, 32, 2), jnp.int32)
  x_hi = jax.lax.bitcast_convert_type(xb[:, 64:].reshape(n, 32, 2), jnp.int32)
  weight_perm = weight[_FULL_PERM, :]

  plo, phi = _sc_segment_sum(n_pad, rows_per_sub, ch)(x_lo, x_hi, src_p, dst_p)
  return _tc_combine_matmul(plo, phi, weight_perm, bias, n)


# speedup vs baseline: 1.2299x; 1.2299x over previous
import jax
import jax.numpy as jnp
from jax import lax
from jax.experimental import pallas as pl
from jax.experimental.pallas import tpu as pltpu
from jax.experimental.pallas import tpu_sc as plsc

NC = 2
NS = 16
CK = 128
NBUF = 2


def _sc_segment_sum(n_pad, rows_per_sub, ch):
  mesh = plsc.VectorSubcoreMesh(core_axis_name="c", subcore_axis_name="s")

  def body(xlo_hbm, xhi_hbm, src_hbm, dst_hbm, outlo_hbm, outhi_hbm,
           zbuf, srcv, dstv, r0, r1, acc, sg0, sg1):
    rows = (r0, r1)
    sg = (sg0, sg1)
    cid = lax.axis_index("c")
    sid = lax.axis_index("s")

    zero16 = jnp.zeros((16,), jnp.float32)
    zrows = rows_per_sub // 8

    def zbody(i, _):
      for j in range(4):
        zbuf[i, pl.ds(j * 16, 16)] = zero16
      return 0

    pltpu.async_copy(src_hbm.at[sid], srcv, sg0)
    pltpu.async_copy(dst_hbm.at[sid], dstv, sg1)
    lax.fori_loop(0, zrows, zbody, 0)
    for j in range(8):
      pltpu.sync_copy(zbuf, acc.at[pl.ds(sid * rows_per_sub + j * zrows, zrows)])
    pltpu.make_async_copy(src_hbm.at[sid], srcv, sg0).wait()
    pltpu.make_async_copy(dst_hbm.at[sid], dstv, sg1).wait()
    plsc.subcore_barrier()

    def run(x_ref):
      for b in range(NBUF):
        pltpu.async_copy(x_ref.at[srcv.at[b]], rows[b], sg[b])

      def gbody(g, _):
        c0 = g * NBUF
        for b in range(NBUF):
          c = c0 + b
          pltpu.make_async_copy(x_ref.at[srcv.at[c]], rows[b], sg[b]).wait()
          pltpu.sync_copy(rows[b], acc.at[dstv.at[c]], add=True)
          pltpu.async_copy(x_ref.at[srcv.at[c + NBUF]], rows[b], sg[b])
        return 0

      lax.fori_loop(0, ch // NBUF - 1, gbody, 0)
      c0 = ch - NBUF
      for b in range(NBUF):
        c = c0 + b
        pltpu.make_async_copy(x_ref.at[srcv.at[c]], rows[b], sg[b]).wait()
        pltpu.sync_copy(rows[b], acc.at[dstv.at[c]], add=True)

    pl.when(cid == 0)(lambda: run(xlo_hbm))
    pl.when(cid == 1)(lambda: run(xhi_hbm))
    plsc.subcore_barrier()

    sl = pl.ds(sid * rows_per_sub, rows_per_sub)
    pl.when(cid == 0)(lambda: pltpu.sync_copy(acc.at[sl], outlo_hbm.at[sl]))
    pl.when(cid == 1)(lambda: pltpu.sync_copy(acc.at[sl], outhi_hbm.at[sl]))

  return pl.kernel(
      body,
      out_type=(
          jax.ShapeDtypeStruct((n_pad, 64), jnp.float32),
          jax.ShapeDtypeStruct((n_pad, 64), jnp.float32),
      ),
      mesh=mesh,
      compiler_params=pltpu.CompilerParams(use_tc_tiling_on_sc=False),
      scratch_types=[
          pltpu.VMEM((rows_per_sub // 8, 64), jnp.float32),
          pltpu.VMEM((ch, CK), jnp.int32),
          pltpu.VMEM((ch, CK), jnp.int32),
          pltpu.VMEM((CK, 64), jnp.float32),
          pltpu.VMEM((CK, 64), jnp.float32),
          pltpu.VMEM_SHARED((n_pad, 64), jnp.float32),
          pltpu.SemaphoreType.DMA,
          pltpu.SemaphoreType.DMA,
      ],
  )


def _tc_body(plo_ref, phi_ref, w_ref, b_ref, o_ref):
  o_ref[...] = (
      jnp.dot(plo_ref[...], w_ref[0:64, :], preferred_element_type=jnp.float32)
      + jnp.dot(phi_ref[...], w_ref[64:128, :], preferred_element_type=jnp.float32)
      + b_ref[...]
  )


def _tc_combine_matmul(plo, phi, weight, bias, n):
  br = 1000
  return pl.pallas_call(
      _tc_body,
      grid=(n // br,),
      in_specs=[
          pl.BlockSpec((br, 64), lambda i: (i, 0)),
          pl.BlockSpec((br, 64), lambda i: (i, 0)),
          pl.BlockSpec((128, 128), lambda i: (0, 0)),
          pl.BlockSpec((1, 128), lambda i: (0, 0)),
      ],
      out_specs=pl.BlockSpec((br, 128), lambda i: (i, 0)),
      out_shape=jax.ShapeDtypeStruct((n, 128), jnp.float32),
  )(plo, phi, weight, bias.reshape(1, 128))


@jax.jit
def kernel(x, edge_index, weight, bias):
  n, d = x.shape
  e = edge_index.shape[1]
  assert d == 128 and weight.shape == (128, 128)

  ch = NBUF * (-(-e // (NS * CK * NBUF)))
  e_pad = NS * ch * CK
  rows_per_sub = 8 * (-(-(n + 1) // (NS * 8)))
  n_pad = rows_per_sub * NS

  src = edge_index[0]
  dst = edge_index[1]
  pad = e_pad - e
  src_p = jnp.concatenate([src, jnp.zeros((pad,), jnp.int32)]).reshape(NS, ch, CK)
  dst_p = jnp.concatenate([dst, jnp.full((pad,), n, jnp.int32)]).reshape(NS, ch, CK)
  x_lo = x[:, :64]
  x_hi = x[:, 64:]

  plo, phi = _sc_segment_sum(n_pad, rows_per_sub, ch)(x_lo, x_hi, src_p, dst_p)
  return _tc_combine_matmul(plo, phi, weight, bias, n)
